# Initial kernel scaffold; baseline (speedup 1.0000x reference)
import jax
import jax.numpy as jnp
from jax.experimental import pallas as pl


def kernel(query, context, fast_keys, fast_vals, deep_keys, deep_vals, W_q, b_q, W_gate, b_gate, mix_logit, W_c1, b_c1, W_c2, b_c2, W_out, b_out):
    raise NotImplementedError("write your pallas kernel here")


# speedup vs baseline: 1.1738x; 1.1738x over previous
import jax
import jax.numpy as jnp
from jax.experimental import pallas as pl
from jax.experimental.pallas import tpu as pltpu

B = 1024
D = 1024
H = 16
S = 256
DH = D // H
S2 = 2 * S
EPS = 1e-8

BB = 256


def _fused(q_ref, c_ref, kt_ref, v_ref,
           wq_ref, bq_ref, wg_ref, bg_ref, mix_ref,
           wc1_ref, bc1_ref, wc2_ref, bc2_ref,
           wo_ref, bo_ref, out_ref):
    x = q_ref[...]
    c = c_ref[...]

    g = jnp.tanh(
        jax.lax.dot_general(c, wg_ref[...], (((1,), (0,)), ((), ())),
                            preferred_element_type=jnp.float32) + bg_ref[...])
    gate = jnp.mean(g, axis=1, keepdims=True)
    mix = jax.nn.sigmoid(mix_ref[0, 0] + gate)

    c1 = jnp.tanh(
        jax.lax.dot_general(c, wc1_ref[...], (((1,), (0,)), ((), ())),
                            preferred_element_type=jnp.float32) + bc1_ref[...])
    conf = jax.nn.sigmoid(
        jnp.sum(c1 * wc2_ref[...], axis=1, keepdims=True) + bc2_ref[0, 0])

    sf = mix * conf
    sd = (1.0 - mix) * conf

    q = jax.lax.dot_general(x, wq_ref[...], (((1,), (0,)), ((), ())),
                            preferred_element_type=jnp.float32) + bq_ref[...]

    kt = kt_ref[...]
    kn = kt / (jnp.sqrt(jnp.sum(kt * kt, axis=1, keepdims=True)) + EPS)

    parts = []
    for h in range(H):
        qh = q[:, h * DH:(h + 1) * DH]
        inv = 1.0 / (jnp.sqrt(jnp.sum(qh * qh, axis=1, keepdims=True)) + EPS)
        sim = jax.lax.dot_general(qh, kn[h], (((1,), (0,)), ((), ())),
                                  preferred_element_type=jnp.float32)
        e = jnp.exp(sim * inv)
        ef = e[:, :S]
        ed = e[:, S:]
        af = ef * (sf / jnp.sum(ef, axis=1, keepdims=True))
        ad = ed * (sd / jnp.sum(ed, axis=1, keepdims=True))
        att = jnp.concatenate([af, ad], axis=1)
        parts.append(
            jax.lax.dot_general(att, v_ref[h], (((1,), (0,)), ((), ())),
                                preferred_element_type=jnp.float32))
    pre = jnp.concatenate(parts, axis=1)

    out_ref[...] = jax.lax.dot_general(
        pre, wo_ref[...], (((1,), (0,)), ((), ())),
        preferred_element_type=jnp.float32) + bo_ref[...]


def kernel(query, context, fast_keys, fast_vals, deep_keys, deep_vals,
           W_q, b_q, W_gate, b_gate, mix_logit, W_c1, b_c1, W_c2, b_c2,
           W_out, b_out):
    keys = jnp.concatenate([fast_keys[0], deep_keys[0]], axis=1)
    kt = jnp.transpose(keys, (0, 2, 1))
    vals = jnp.concatenate([fast_vals[0], deep_vals[0]], axis=1)

    grid = (B // BB,)
    row_spec = pl.BlockSpec((BB, D), lambda i: (i, 0))

    def full(shape):
        return pl.BlockSpec(shape, lambda i: (0,) * len(shape))

    out = pl.pallas_call(
        _fused,
        grid=grid,
        in_specs=[
            row_spec,
            row_spec,
            full((H, DH, S2)),
            full((H, S2, DH)),
            full((D, D)),
            full((1, D)),
            full((D, D)),
            full((1, D)),
            full((1, 1)),
            full((D, D)),
            full((1, D)),
            full((1, D)),
            full((1, 1)),
            full((D, D)),
            full((1, D)),
        ],
        out_specs=row_spec,
        out_shape=jax.ShapeDtypeStruct((B, D), jnp.float32),
    )(
        query, context, kt, vals,
        W_q.T, b_q.reshape(1, D),
        W_gate.T, b_gate.reshape(1, D),
        mix_logit.reshape(1, 1),
        W_c1.T, b_c1.reshape(1, D),
        W_c2.reshape(1, D), b_c2.reshape(1, 1),
        W_out.T, b_out.reshape(1, D),
    )
    return out


# speedup vs baseline: 1.2591x; 1.0727x over previous
import jax
import jax.numpy as jnp
from jax.experimental import pallas as pl
from jax.experimental.pallas import tpu as pltpu

B = 1024
D = 1024
H = 16
S = 256
DH = D // H
S2 = 2 * S
EPS = 1e-8

BB = 256


def _mm(a, b):
    return jax.lax.dot_general(a, b, (((1,), (0,)), ((), ())),
                               preferred_element_type=jnp.float32)


def _fused(q_ref, c_ref, kt_ref, v_ref,
           wq_ref, bq_ref, wg_ref, bg_ref, mix_ref,
           wc1_ref, bc1_ref, wc2_ref, bc2_ref,
           wo_ref, bo_ref, out_ref):
    x = q_ref[...]
    c = c_ref[...]

    g = jnp.tanh(_mm(c, wg_ref[...]) + bg_ref[...])
    gate = jnp.mean(g, axis=1, keepdims=True)
    mix = jax.nn.sigmoid(mix_ref[0, 0] + gate)

    c1 = jnp.tanh(_mm(c, wc1_ref[...]) + bc1_ref[...])
    conf = jax.nn.sigmoid(
        jnp.sum(c1 * wc2_ref[...], axis=1, keepdims=True) + bc2_ref[0, 0])

    sf = mix * conf
    sd = (1.0 - mix) * conf

    q = _mm(x, wq_ref[...]) + bq_ref[...]

    kt = kt_ref[...]
    kn = (kt / (jnp.sqrt(jnp.sum(kt * kt, axis=1, keepdims=True)) + EPS)
          ).astype(jnp.bfloat16)

    qb = q.astype(jnp.bfloat16)
    parts = []
    for h in range(H):
        qh = q[:, h * DH:(h + 1) * DH]
        inv = 1.0 / (jnp.sqrt(jnp.sum(qh * qh, axis=1, keepdims=True)) + EPS)
        sim = _mm(qb[:, h * DH:(h + 1) * DH], kn[h])
        e = jnp.exp(sim * inv)
        ef = e[:, :S]
        ed = e[:, S:]
        af = ef * (sf / jnp.sum(ef, axis=1, keepdims=True))
        ad = ed * (sd / jnp.sum(ed, axis=1, keepdims=True))
        att = jnp.concatenate([af, ad], axis=1).astype(jnp.bfloat16)
        parts.append(_mm(att, v_ref[h]))
    pre = jnp.concatenate(parts, axis=1).astype(jnp.bfloat16)

    out_ref[...] = _mm(pre, wo_ref[...]) + bo_ref[...]


def kernel(query, context, fast_keys, fast_vals, deep_keys, deep_vals,
           W_q, b_q, W_gate, b_gate, mix_logit, W_c1, b_c1, W_c2, b_c2,
           W_out, b_out):
    keys = jnp.concatenate([fast_keys[0], deep_keys[0]], axis=1)
    kt = jnp.transpose(keys, (0, 2, 1))
    vals = jnp.concatenate([fast_vals[0], deep_vals[0]],
                           axis=1).astype(jnp.bfloat16)
    bf = jnp.bfloat16

    grid = (B // BB,)
    row_spec = pl.BlockSpec((BB, D), lambda i: (i, 0))

    def full(shape):
        return pl.BlockSpec(shape, lambda i: (0,) * len(shape))

    out = pl.pallas_call(
        _fused,
        grid=grid,
        in_specs=[
            row_spec,
            row_spec,
            full((H, DH, S2)),
            full((H, S2, DH)),
            full((D, D)),
            full((1, D)),
            full((D, D)),
            full((1, D)),
            full((1, 1)),
            full((D, D)),
            full((1, D)),
            full((1, D)),
            full((1, 1)),
            full((D, D)),
            full((1, D)),
        ],
        out_specs=row_spec,
        out_shape=jax.ShapeDtypeStruct((B, D), jnp.float32),
    )(
        query.astype(bf), context.astype(bf), kt, vals,
        W_q.T.astype(bf), b_q.reshape(1, D),
        W_gate.T.astype(bf), b_gate.reshape(1, D),
        mix_logit.reshape(1, 1),
        W_c1.T.astype(bf), b_c1.reshape(1, D),
        W_c2.reshape(1, D), b_c2.reshape(1, 1),
        W_out.T.astype(bf), b_out.reshape(1, D),
    )
    return out


# speedup vs baseline: 1.3774x; 1.0939x over previous
import jax
import jax.numpy as jnp
from jax.experimental import pallas as pl
from jax.experimental.pallas import tpu as pltpu

B = 1024
D = 1024
H = 16
S = 256
DH = D // H
S2 = 2 * S
EPS = 1e-8

BB = 256


def _mmt(a, b):
    return jax.lax.dot_general(a, b, (((1,), (1,)), ((), ())),
                               preferred_element_type=jnp.float32)


def _fused(q_ref, c_ref, fk_ref, fv_ref, dk_ref, dv_ref,
           wq_ref, bq_ref, wg_ref, bg_ref, mix_ref,
           wc1_ref, bc1_ref, wc2_ref, bc2_ref,
           wo_ref, bo_ref, out_ref):
    bf = jnp.bfloat16
    x = q_ref[...].astype(bf)
    c = c_ref[...].astype(bf)

    g = jnp.tanh(_mmt(c, wg_ref[...].astype(bf)) + bg_ref[...])
    gate = jnp.mean(g, axis=1, keepdims=True)
    mix = jax.nn.sigmoid(mix_ref[0, 0] + gate)

    c1 = jnp.tanh(_mmt(c, wc1_ref[...].astype(bf)) + bc1_ref[...])
    conf = jax.nn.sigmoid(
        jnp.sum(c1 * wc2_ref[...], axis=1, keepdims=True) + bc2_ref[0, 0])

    sf = mix * conf
    sd = (1.0 - mix) * conf

    q = _mmt(x, wq_ref[...].astype(bf)) + bq_ref[...]

    fk = fk_ref[0]
    dk = dk_ref[0]
    fkn = (fk / (jnp.sqrt(jnp.sum(fk * fk, axis=2, keepdims=True)) + EPS)
           ).astype(bf)
    dkn = (dk / (jnp.sqrt(jnp.sum(dk * dk, axis=2, keepdims=True)) + EPS)
           ).astype(bf)
    fv = fv_ref[0].astype(bf)
    dv = dv_ref[0].astype(bf)

    qb = q.astype(bf)
    parts = []
    for h in range(H):
        qh = q[:, h * DH:(h + 1) * DH]
        inv = 1.0 / (jnp.sqrt(jnp.sum(qh * qh, axis=1, keepdims=True)) + EPS)
        qhb = qb[:, h * DH:(h + 1) * DH]
        simf = _mmt(qhb, fkn[h])
        simd = _mmt(qhb, dkn[h])
        ef = jnp.exp(simf * inv)
        ed = jnp.exp(simd * inv)
        af = (ef * (sf / jnp.sum(ef, axis=1, keepdims=True))).astype(bf)
        ad = (ed * (sd / jnp.sum(ed, axis=1, keepdims=True))).astype(bf)
        vf = jax.lax.dot_general(af, fv[h], (((1,), (0,)), ((), ())),
                                 preferred_element_type=jnp.float32)
        vd = jax.lax.dot_general(ad, dv[h], (((1,), (0,)), ((), ())),
                                 preferred_element_type=jnp.float32)
        parts.append(vf + vd)
    pre = jnp.concatenate(parts, axis=1).astype(bf)

    out_ref[...] = _mmt(pre, wo_ref[...].astype(bf)) + bo_ref[...]


def kernel(query, context, fast_keys, fast_vals, deep_keys, deep_vals,
           W_q, b_q, W_gate, b_gate, mix_logit, W_c1, b_c1, W_c2, b_c2,
           W_out, b_out):
    grid = (B // BB,)
    row_spec = pl.BlockSpec((BB, D), lambda i: (i, 0))

    def full(shape):
        return pl.BlockSpec(shape, lambda i: (0,) * len(shape))

    out = pl.pallas_call(
        _fused,
        grid=grid,
        in_specs=[
            row_spec,
            row_spec,
            full((1, H, S, DH)),
            full((1, H, S, DH)),
            full((1, H, S, DH)),
            full((1, H, S, DH)),
            full((D, D)),
            full((1, D)),
            full((D, D)),
            full((1, D)),
            full((1, 1)),
            full((D, D)),
            full((1, D)),
            full((1, D)),
            full((1, 1)),
            full((D, D)),
            full((1, D)),
        ],
        out_specs=row_spec,
        out_shape=jax.ShapeDtypeStruct((B, D), jnp.float32),
    )(
        query, context, fast_keys, fast_vals, deep_keys, deep_vals,
        W_q, b_q.reshape(1, D),
        W_gate, b_gate.reshape(1, D),
        mix_logit.reshape(1, 1),
        W_c1, b_c1.reshape(1, D),
        W_c2, b_c2.reshape(1, 1),
        W_out, b_out.reshape(1, D),
    )
    return out


# speedup vs baseline: 1.5053x; 1.0928x over previous
import jax
import jax.numpy as jnp
from jax.experimental import pallas as pl
from jax.experimental.pallas import tpu as pltpu

B = 1024
D = 1024
H = 16
S = 256
DH = D // H
S2 = 2 * S
EPS = 1e-8

BB = 256


def _mmt(a, b):
    return jax.lax.dot_general(a, b, (((1,), (1,)), ((), ())),
                               preferred_element_type=jnp.float32)


def _fused(q_ref, c_ref, fk_ref, fv_ref, dk_ref, dv_ref,
           wq_ref, bq_ref, wg_ref, bg_ref, mix_ref,
           wc1_ref, bc1_ref, wc2_ref, bc2_ref,
           wo_ref, bo_ref, out_ref,
           wqb, wgb, wc1b, wob, fkn_s, dkn_s, fvb, dvb):
    bf = jnp.bfloat16

    @pl.when(pl.program_id(0) == 0)
    def _prep():
        wqb[...] = wq_ref[...].astype(bf)
        wgb[...] = wg_ref[...].astype(bf)
        wc1b[...] = wc1_ref[...].astype(bf)
        wob[...] = wo_ref[...].astype(bf)
        fk = fk_ref[...]
        dk = dk_ref[...]
        fkn_s[...] = (fk / (jnp.sqrt(jnp.sum(fk * fk, axis=2, keepdims=True))
                            + EPS)).astype(bf)
        dkn_s[...] = (dk / (jnp.sqrt(jnp.sum(dk * dk, axis=2, keepdims=True))
                            + EPS)).astype(bf)
        fvb[...] = fv_ref[...].astype(bf)
        dvb[...] = dv_ref[...].astype(bf)

    x = q_ref[...].astype(bf)
    c = c_ref[...].astype(bf)

    g = jnp.tanh(_mmt(c, wgb[...]) + bg_ref[...])
    gate = jnp.mean(g, axis=1, keepdims=True)
    mix = jax.nn.sigmoid(mix_ref[0, 0] + gate)

    c1 = jnp.tanh(_mmt(c, wc1b[...]) + bc1_ref[...])
    conf = jax.nn.sigmoid(
        jnp.sum(c1 * wc2_ref[...], axis=1, keepdims=True) + bc2_ref[0, 0])

    sf = mix * conf
    sd = (1.0 - mix) * conf

    qb = (_mmt(x, wqb[...]) + bq_ref[...]).astype(bf)

    parts = []
    for h in range(H):
        qhb = qb[:, h * DH:(h + 1) * DH]
        qh32 = qhb.astype(jnp.float32)
        inv = 1.0 / (jnp.sqrt(jnp.sum(qh32 * qh32, axis=1, keepdims=True))
                     + EPS)
        simf = _mmt(qhb, fkn_s[h])
        simd = _mmt(qhb, dkn_s[h])
        ef = jnp.exp(simf * inv)
        ed = jnp.exp(simd * inv)
        af = (ef * (sf / jnp.sum(ef, axis=1, keepdims=True))).astype(bf)
        ad = (ed * (sd / jnp.sum(ed, axis=1, keepdims=True))).astype(bf)
        vf = jax.lax.dot_general(af, fvb[h], (((1,), (0,)), ((), ())),
                                 preferred_element_type=jnp.float32)
        vd = jax.lax.dot_general(ad, dvb[h], (((1,), (0,)), ((), ())),
                                 preferred_element_type=jnp.float32)
        parts.append(vf + vd)
    pre = jnp.concatenate(parts, axis=1).astype(bf)

    out_ref[...] = _mmt(pre, wob[...]) + bo_ref[...]


def kernel(query, context, fast_keys, fast_vals, deep_keys, deep_vals,
           W_q, b_q, W_gate, b_gate, mix_logit, W_c1, b_c1, W_c2, b_c2,
           W_out, b_out):
    grid = (B // BB,)
    row_spec = pl.BlockSpec((BB, D), lambda i: (i, 0))

    def full(shape):
        return pl.BlockSpec(shape, lambda i: (0,) * len(shape))

    bf = jnp.bfloat16
    out = pl.pallas_call(
        _fused,
        grid=grid,
        in_specs=[
            row_spec,
            row_spec,
            full((H, S, DH)),
            full((H, S, DH)),
            full((H, S, DH)),
            full((H, S, DH)),
            full((D, D)),
            full((1, D)),
            full((D, D)),
            full((1, D)),
            full((1, 1)),
            full((D, D)),
            full((1, D)),
            full((1, D)),
            full((1, 1)),
            full((D, D)),
            full((1, D)),
        ],
        out_specs=row_spec,
        out_shape=jax.ShapeDtypeStruct((B, D), jnp.float32),
        scratch_shapes=[
            pltpu.VMEM((D, D), bf),
            pltpu.VMEM((D, D), bf),
            pltpu.VMEM((D, D), bf),
            pltpu.VMEM((D, D), bf),
            pltpu.VMEM((H, S, DH), bf),
            pltpu.VMEM((H, S, DH), bf),
            pltpu.VMEM((H, S, DH), bf),
            pltpu.VMEM((H, S, DH), bf),
        ],
    )(
        query, context,
        fast_keys.reshape(H, S, DH), fast_vals.reshape(H, S, DH),
        deep_keys.reshape(H, S, DH), deep_vals.reshape(H, S, DH),
        W_q, b_q.reshape(1, D),
        W_gate, b_gate.reshape(1, D),
        mix_logit.reshape(1, 1),
        W_c1, b_c1.reshape(1, D),
        W_c2, b_c2.reshape(1, 1),
        W_out, b_out.reshape(1, D),
    )
    return out


# speedup vs baseline: 1.6729x; 1.1114x over previous
import jax
import jax.numpy as jnp
from jax.experimental import pallas as pl
from jax.experimental.pallas import tpu as pltpu

B = 1024
D = 1024
H = 16
S = 256
DH = D // H
EPS = 1e-8

BB = 256


def _mmt(a, b):
    return jax.lax.dot_general(a, b, (((1,), (1,)), ((), ())),
                               preferred_element_type=jnp.float32)


def _mm(a, b):
    return jax.lax.dot_general(a, b, (((1,), (0,)), ((), ())),
                               preferred_element_type=jnp.float32)


def _fused(q_ref, c_ref, fk_ref, fv_ref, dk_ref, dv_ref,
           wq_ref, bq_ref, wg_ref, bg_ref, mix_ref,
           wc1_ref, bc1_ref, wc2_ref, bc2_ref,
           wo_ref, bo_ref, out_ref,
           wqb, wgb, wc1b, wob, fkn_s, dkn_s, fva, dva, m_s):
    bf = jnp.bfloat16

    @pl.when(pl.program_id(0) == 0)
    def _prep():
        wqb[...] = wq_ref[...].astype(bf)
        wgb[...] = wg_ref[...].astype(bf)
        wc1b[...] = wc1_ref[...].astype(bf)
        wob[...] = wo_ref[...].astype(bf)
        fk = fk_ref[...]
        dk = dk_ref[...]
        fkn_s[...] = (fk / (jnp.sqrt(jnp.sum(fk * fk, axis=2, keepdims=True))
                            + EPS)).astype(bf)
        dkn_s[...] = (dk / (jnp.sqrt(jnp.sum(dk * dk, axis=2, keepdims=True))
                            + EPS)).astype(bf)
        ones_col = jnp.ones((H, S, 1), dtype=bf)
        fva[...] = jnp.concatenate([fv_ref[...].astype(bf), ones_col], axis=2)
        dva[...] = jnp.concatenate([dv_ref[...].astype(bf), ones_col], axis=2)
        di = jax.lax.broadcasted_iota(jnp.int32, (D, H), 0) // DH
        hi = jax.lax.broadcasted_iota(jnp.int32, (D, H), 1)
        m_s[...] = (di == hi).astype(jnp.float32)

    x = q_ref[...].astype(bf)
    c = c_ref[...].astype(bf)

    g = jnp.tanh(_mmt(c, wgb[...]) + bg_ref[...])
    gate = jnp.mean(g, axis=1, keepdims=True)
    mix = jax.nn.sigmoid(mix_ref[0, 0] + gate)

    c1 = jnp.tanh(_mmt(c, wc1b[...]) + bc1_ref[...])
    conf = jax.nn.sigmoid(
        jnp.sum(c1 * wc2_ref[...], axis=1, keepdims=True) + bc2_ref[0, 0])

    sf = mix * conf
    sd = (1.0 - mix) * conf

    q = _mmt(x, wqb[...]) + bq_ref[...]
    n2 = _mm(q * q, m_s[...])
    inv = 1.0 / (jnp.sqrt(n2) + EPS)
    inv_exp = _mmt(inv, m_s[...])
    qs = (q * inv_exp).astype(bf)

    parts = []
    for h in range(H):
        qh = qs[:, h * DH:(h + 1) * DH]
        ef = jnp.exp(_mmt(qh, fkn_s[h])).astype(bf)
        ed = jnp.exp(_mmt(qh, dkn_s[h])).astype(bf)
        vf = _mm(ef, fva[h])
        vd = _mm(ed, dva[h])
        scf = sf / vf[:, DH:DH + 1]
        scd = sd / vd[:, DH:DH + 1]
        parts.append((vf[:, :DH] * scf + vd[:, :DH] * scd).astype(bf))
    pre = jnp.concatenate(parts, axis=1)

    out_ref[...] = _mmt(pre, wob[...]) + bo_ref[...]


def kernel(query, context, fast_keys, fast_vals, deep_keys, deep_vals,
           W_q, b_q, W_gate, b_gate, mix_logit, W_c1, b_c1, W_c2, b_c2,
           W_out, b_out):
    grid = (B // BB,)
    row_spec = pl.BlockSpec((BB, D), lambda i: (i, 0))

    def full(shape):
        return pl.BlockSpec(shape, lambda i: (0,) * len(shape))

    bf = jnp.bfloat16
    out = pl.pallas_call(
        _fused,
        grid=grid,
        in_specs=[
            row_spec,
            row_spec,
            full((H, S, DH)),
            full((H, S, DH)),
            full((H, S, DH)),
            full((H, S, DH)),
            full((D, D)),
            full((1, D)),
            full((D, D)),
            full((1, D)),
            full((1, 1)),
            full((D, D)),
            full((1, D)),
            full((1, D)),
            full((1, 1)),
            full((D, D)),
            full((1, D)),
        ],
        out_specs=row_spec,
        out_shape=jax.ShapeDtypeStruct((B, D), jnp.float32),
        scratch_shapes=[
            pltpu.VMEM((D, D), bf),
            pltpu.VMEM((D, D), bf),
            pltpu.VMEM((D, D), bf),
            pltpu.VMEM((D, D), bf),
            pltpu.VMEM((H, S, DH), bf),
            pltpu.VMEM((H, S, DH), bf),
            pltpu.VMEM((H, S, DH + 1), bf),
            pltpu.VMEM((H, S, DH + 1), bf),
            pltpu.VMEM((D, H), jnp.float32),
        ],
    )(
        query, context,
        fast_keys.reshape(H, S, DH), fast_vals.reshape(H, S, DH),
        deep_keys.reshape(H, S, DH), deep_vals.reshape(H, S, DH),
        W_q, b_q.reshape(1, D),
        W_gate, b_gate.reshape(1, D),
        mix_logit.reshape(1, 1),
        W_c1, b_c1.reshape(1, D),
        W_c2, b_c2.reshape(1, 1),
        W_out, b_out.reshape(1, D),
    )
    return out
